# baseline (speedup 1.0000x reference)
import functools

import jax
import jax.numpy as jnp
from jax import lax
from jax.experimental import pallas as pl
from jax.experimental.pallas import tpu as pltpu
from jax.experimental.pallas import tpu_sc as plsc

_NC = 2
_NS = 16
_NW = _NC * _NS
_L = 16
_MAP = 64


def _build(n):
    per_w = n // _NW
    mesh = plsc.VectorSubcoreMesh(
        core_axis_name="c", subcore_axis_name="s",
        num_cores=_NC, num_subcores=_NS)

    @functools.partial(
        pl.kernel,
        out_type=(jax.ShapeDtypeStruct((n,), jnp.int32),
                  jax.ShapeDtypeStruct((n,), jnp.int32)),
        mesh=mesh,
        compiler_params=pltpu.CompilerParams(
            needs_layout_passes=False,
            use_tc_tiling_on_sc=False,
            disable_bounds_checks=True,
            disable_semaphore_checks=True,
            skip_device_barrier=True),
        scratch_types=[
            pltpu.VMEM((per_w,), jnp.int32),
            pltpu.VMEM((per_w,), jnp.int32),
            pltpu.VMEM((per_w,), jnp.int32),
            pltpu.VMEM((128,), jnp.int32),
            pltpu.VMEM((128,), jnp.int32),
            pltpu.VMEM((128,), jnp.int32),
            pltpu.SemaphoreType.DMA,
            pltpu.SemaphoreType.DMA,
        ],
    )
    def dispatch(idx_hbm, devmap_hbm, locmap_hbm, dev_hbm, loc_hbm,
                 idx_v, dev_v, loc_v, devmap_v, locmap_v, packed_v,
                 sem_in, sem_out):
        wid = lax.axis_index("s") * _NC + lax.axis_index("c")
        base = wid * per_w
        cp_idx = pltpu.async_copy(
            idx_hbm.at[pl.ds(base, per_w)], idx_v, sem_in)
        cp_dm = pltpu.async_copy(devmap_hbm, devmap_v.at[pl.ds(0, _MAP)],
                                 sem_out)
        cp_lm = pltpu.async_copy(locmap_hbm, locmap_v.at[pl.ds(0, _MAP)],
                                 sem_out)
        cp_dm.wait()
        cp_lm.wait()

        for j in range(_MAP // _L):
            sl = pl.ds(j * _L, _L)
            packed_v[sl] = (devmap_v[sl] << 16) | (locmap_v[sl] & 0xFFFF)
        cp_idx.wait()

        n_chunks = 2
        chunk = per_w // n_chunks
        out_cps = []
        for c in range(n_chunks):
            lo = c * chunk

            @plsc.parallel_loop(lo, lo + chunk, _L, unroll=8)
            def _(off):
                sl = pl.ds(off, _L)
                g = plsc.load_gather(packed_v, [idx_v[sl]])
                dev_v[sl] = g >> 16
                loc_v[sl] = (g << 16) >> 16

            out_cps.append(pltpu.async_copy(
                dev_v.at[pl.ds(lo, chunk)],
                dev_hbm.at[pl.ds(base + lo, chunk)], sem_out))
            out_cps.append(pltpu.async_copy(
                loc_v.at[pl.ds(lo, chunk)],
                loc_hbm.at[pl.ds(base + lo, chunk)], sem_out))
        for cp in out_cps:
            cp.wait()

    return dispatch


def kernel(indices_expert, weight1, weight2, device_indices_map,
           local_expert_indices_map):
    t, k = indices_expert.shape
    n = t * k
    x = indices_expert.astype(jnp.int32)
    tt = t // 128
    flat = x.reshape(tt, 128, k).transpose(0, 2, 1).reshape(n)
    dev, loc = _build(n)(flat,
                         device_indices_map.astype(jnp.int32),
                         local_expert_indices_map.astype(jnp.int32))
    def unflat(a):
        return a.reshape(tt, k, 128).transpose(0, 2, 1).reshape(t, k)
    out_dtype = indices_expert.dtype
    return unflat(dev).astype(out_dtype), unflat(loc).astype(out_dtype)


# speedup vs baseline: 1.0011x; 1.0011x over previous
import functools

import jax
import jax.numpy as jnp
from jax import lax
from jax.experimental import pallas as pl
from jax.experimental.pallas import tpu as pltpu
from jax.experimental.pallas import tpu_sc as plsc

_NC = 2
_NS = 16
_NW = _NC * _NS
_L = 16
_MAP = 64


def _build(n):
    per_w = n // _NW
    mesh = plsc.VectorSubcoreMesh(
        core_axis_name="c", subcore_axis_name="s",
        num_cores=_NC, num_subcores=_NS)

    @functools.partial(
        pl.kernel,
        out_type=(jax.ShapeDtypeStruct((n,), jnp.int32),
                  jax.ShapeDtypeStruct((n,), jnp.int32)),
        mesh=mesh,
        compiler_params=pltpu.CompilerParams(
            needs_layout_passes=False,
            use_tc_tiling_on_sc=False,
            disable_bounds_checks=True,
            disable_semaphore_checks=True,
            skip_device_barrier=True),
        scratch_types=[
            pltpu.VMEM((per_w,), jnp.int32),
            pltpu.VMEM((per_w,), jnp.int32),
            pltpu.VMEM((per_w,), jnp.int32),
            pltpu.VMEM((128,), jnp.int32),
            pltpu.VMEM((128,), jnp.int32),
            pltpu.VMEM((128,), jnp.int32),
            pltpu.SemaphoreType.DMA,
            pltpu.SemaphoreType.DMA,
        ],
    )
    def dispatch(idx_hbm, devmap_hbm, locmap_hbm, dev_hbm, loc_hbm,
                 idx_v, dev_v, loc_v, devmap_v, locmap_v, packed_v,
                 sem_in, sem_out):
        wid = lax.axis_index("s") * _NC + lax.axis_index("c")
        base = wid * per_w
        cp_idx = pltpu.async_copy(
            idx_hbm.at[pl.ds(base, per_w)], idx_v, sem_in)
        cp_dm = pltpu.async_copy(devmap_hbm, devmap_v.at[pl.ds(0, _MAP)],
                                 sem_out)
        cp_lm = pltpu.async_copy(locmap_hbm, locmap_v.at[pl.ds(0, _MAP)],
                                 sem_out)
        cp_dm.wait()
        cp_lm.wait()

        for j in range(_MAP // _L):
            sl = pl.ds(j * _L, _L)
            packed_v[sl] = (devmap_v[sl] << 16) | (locmap_v[sl] & 0xFFFF)
        cp_idx.wait()

        n_chunks = 2
        chunk = per_w // n_chunks
        out_cps = []
        for c in range(n_chunks):
            lo = c * chunk

            @plsc.parallel_loop(lo, lo + chunk, _L, unroll=8)
            def _(off):
                sl = pl.ds(off, _L)
                g = plsc.load_gather(packed_v, [idx_v[sl]])
                dev_v[sl] = g >> 16
                loc_v[sl] = (g << 16) >> 16

            out_cps.append(pltpu.async_copy(
                dev_v.at[pl.ds(lo, chunk)],
                dev_hbm.at[pl.ds(base + lo, chunk)], sem_out))
            out_cps.append(pltpu.async_copy(
                loc_v.at[pl.ds(lo, chunk)],
                loc_hbm.at[pl.ds(base + lo, chunk)], sem_out))
        for cp in out_cps:
            cp.wait()

    return dispatch


def kernel(indices_expert, weight1, weight2, device_indices_map,
           local_expert_indices_map):
    t, k = indices_expert.shape
    n = t * k
    x = indices_expert.astype(jnp.int32)
    phys = t % 128 == 0
    tt = t // 128
    if phys:
        flat = x.reshape(tt, 128, k).transpose(0, 2, 1).reshape(n)
    else:
        flat = x.reshape(n)
    dev, loc = _build(n)(flat,
                         device_indices_map.astype(jnp.int32),
                         local_expert_indices_map.astype(jnp.int32))
    def unflat(a):
        if phys:
            return a.reshape(tt, k, 128).transpose(0, 2, 1).reshape(t, k)
        return a.reshape(t, k)
    out_dtype = indices_expert.dtype
    return unflat(dev).astype(out_dtype), unflat(loc).astype(out_dtype)
